# Initial kernel scaffold; baseline (speedup 1.0000x reference)
import jax
import jax.numpy as jnp
from jax.experimental import pallas as pl


def kernel(y, x):
    raise NotImplementedError("write your pallas kernel here")


# speedup vs baseline: 482.4971x; 482.4971x over previous
import functools
import math

import jax
import jax.numpy as jnp
import numpy as np
from jax.experimental import pallas as pl

SR = 44100
NBINS = 128
HOP = 512
FMIN = 100.0
FMAX = 12800.0

BATCH = 4
NSAMP = 88200
T = 173
TPAD = 176
NB = 126
LANES = 128
NCHUNK = 32
COLS = BATCH * TPAD
CCHUNK = 64
NCC = COLS // CCHUNK


def _make_consts():
    num_octaves = np.log2(FMAX / FMIN)
    bpo = int(NBINS / num_octaves)
    Q = 1.0 / (2.0 ** (1.0 / bpo) - 1.0)
    n_bins = int(np.ceil(bpo * np.log2(FMAX / FMIN)))
    freqs = FMIN * 2.0 ** (np.arange(n_bins, dtype=np.float64) / bpo)
    fft_len = int(2 ** np.ceil(np.log2(np.ceil(Q * SR / FMIN))))
    lengths = np.ceil(Q * SR / freqs)
    kr = np.zeros((n_bins, fft_len), dtype=np.float32)
    ki = np.zeros((n_bins, fft_len), dtype=np.float32)
    for k in range(n_bins):
        l = int(lengths[k])
        if l % 2 == 1:
            start = int(np.ceil(fft_len / 2.0 - l / 2.0)) - 1
        else:
            start = int(np.ceil(fft_len / 2.0 - l / 2.0))
        n = np.arange(l)
        win = 0.5 - 0.5 * np.cos(2.0 * np.pi * n / l)
        r = np.arange(-l // 2, -l // 2 + l)
        sig = (win / l) * np.exp(1j * 2.0 * np.pi * freqs[k] * r / SR)
        kr[k, start:start + l] = sig.real.astype(np.float32)
        ki[k, start:start + l] = sig.imag.astype(np.float32)
    krt = np.zeros((NCHUNK, HOP, LANES), dtype=np.float32)
    kit = np.zeros((NCHUNK, HOP, LANES), dtype=np.float32)
    krt[:, :, :n_bins] = kr.reshape(n_bins, NCHUNK, HOP).transpose(1, 2, 0)
    kit[:, :, :n_bins] = ki.reshape(n_bins, NCHUNK, HOP).transpose(1, 2, 0)
    sql = np.zeros((1, LANES), dtype=np.float32)
    sql[0, :n_bins] = np.sqrt(lengths).astype(np.float32)
    cum = np.zeros((LANES, LANES), dtype=np.float32)
    for k in range(n_bins):
        cum[k, k:n_bins] = 1.0
    f = (freqs / SR).astype(np.float64)
    w = np.zeros((1, LANES), dtype=np.float32)
    w[0, :n_bins - 1] = (f[1:] - f[:-1]).astype(np.float32)
    wk = w[0, :n_bins - 1].astype(np.float64)
    g = np.zeros((8, COLS), dtype=np.float32)
    for b in range(BATCH):
        g[b, b * TPAD:b * TPAD + T] = 100.0 / T
    g = g.reshape(8, NCC, CCHUNK).transpose(1, 0, 2)
    return (jnp.asarray(krt), jnp.asarray(kit), jnp.asarray(sql),
            jnp.asarray(cum), jnp.asarray(w), [float(v) for v in wk],
            jnp.asarray(g), fft_len)


_KRT, _KIT, _SQL, _CUM, _W, _WK, _G, _FFTLEN = _make_consts()
_PAD = _FFTLEN // 2
_XROWS = 208


def _cdf_body(xs_ref, krt_ref, kit_ref, sql_ref, cum_ref, out_ref):
    acc_r = jnp.zeros((TPAD, LANES), jnp.float32)
    acc_i = jnp.zeros((TPAD, LANES), jnp.float32)
    dn = (((1,), (0,)), ((), ()))
    for c in range(NCHUNK):
        xc = xs_ref[0, c:c + TPAD, :]
        acc_r += jax.lax.dot_general(xc, krt_ref[c], dn,
                                     preferred_element_type=jnp.float32)
        acc_i += jax.lax.dot_general(xc, kit_ref[c], dn,
                                     preferred_element_type=jnp.float32)
    mag = jnp.sqrt(acc_r * acc_r + acc_i * acc_i) * sql_ref[...]
    fx = jnp.log(mag + 1.0)
    F = jax.lax.dot_general(fx, cum_ref[...], dn,
                            preferred_element_type=jnp.float32)
    A = F / F[:, NB - 1:NB]
    ti = jax.lax.broadcasted_iota(jnp.int32, (TPAD, 1), 0)
    out_ref[0] = jnp.where(ti < T, A, 0.0)


def _ot_body(ab_ref, w_ref, g_ref, out_ref):
    j = pl.program_id(0)
    a = ab_ref[0]
    b = ab_ref[1]
    sab = jnp.zeros((CCHUNK, LANES), jnp.float32)
    saa = jnp.zeros((CCHUNK, LANES), jnp.float32)
    sbb = jnp.zeros((CCHUNK, LANES), jnp.float32)
    for k in range(NB - 1):
        wk = _WK[k]
        ak = a[:, k:k + 1]
        bk = b[:, k:k + 1]
        sab += wk * jnp.abs(ak - b)
        saa += wk * jnp.abs(ak - a)
        sbb += wk * jnp.abs(bk - b)
    r = (sab - 0.5 * (saa + sbb)) * w_ref[...]
    col = jnp.sum(r, axis=1, keepdims=True)
    part = jax.lax.dot_general(g_ref[0], col, (((1,), (0,)), ((), ())),
                               preferred_element_type=jnp.float32)

    @pl.when(j == 0)
    def _():
        out_ref[...] = jnp.zeros_like(out_ref)

    out_ref[...] += part


@jax.jit
def kernel(y, x):
    def frames(sig):
        xp = jnp.pad(sig, ((0, 0), (_PAD, _PAD)), mode='reflect')
        xp = jnp.pad(xp, ((0, 0), (0, _XROWS * HOP - xp.shape[1])))
        return xp.reshape(BATCH, _XROWS, HOP)

    xs = jnp.concatenate([frames(x), frames(y)], axis=0)

    cdf = pl.pallas_call(
        _cdf_body,
        grid=(2 * BATCH,),
        in_specs=[
            pl.BlockSpec((1, _XROWS, HOP), lambda i: (i, 0, 0)),
            pl.BlockSpec((NCHUNK, HOP, LANES), lambda i: (0, 0, 0)),
            pl.BlockSpec((NCHUNK, HOP, LANES), lambda i: (0, 0, 0)),
            pl.BlockSpec((1, LANES), lambda i: (0, 0)),
            pl.BlockSpec((LANES, LANES), lambda i: (0, 0)),
        ],
        out_specs=pl.BlockSpec((1, TPAD, LANES), lambda i: (i, 0, 0)),
        out_shape=jax.ShapeDtypeStruct((2 * BATCH, TPAD, LANES),
                                       jnp.float32),
    )(xs, _KRT, _KIT, _SQL, _CUM)

    ab = cdf.reshape(2, COLS, LANES)

    out = pl.pallas_call(
        _ot_body,
        grid=(NCC,),
        in_specs=[
            pl.BlockSpec((2, CCHUNK, LANES), lambda j: (0, j, 0)),
            pl.BlockSpec((1, LANES), lambda j: (0, 0)),
            pl.BlockSpec((1, 8, CCHUNK), lambda j: (j, 0, 0)),
        ],
        out_specs=pl.BlockSpec((8, 1), lambda j: (0, 0)),
        out_shape=jax.ShapeDtypeStruct((8, 1), jnp.float32),
    )(ab, _W, _G)

    return out[:BATCH, 0]
